# baseline (speedup 1.0000x reference)
import numpy as np

import jax
import jax.numpy as jnp
from jax.experimental import pallas as pl
from jax.experimental.pallas import tpu as pltpu


def _fused_cnn(x, w1, b1, w2, b2, fw1, fb1, fw2, fb2, *, tb):
    C, H, W, B = x.shape
    C1 = w1.shape[0]
    C2 = w2.shape[0]
    OH1 = (H - 8) // 4 + 1
    OW1 = (W - 8) // 4 + 1
    OH2 = (OH1 - 4) // 2 + 1
    OW2 = (OW1 - 4) // 2 + 1
    HID = fw1.shape[0]
    NP = fw2.shape[0]
    K1 = C * 64

    HCH = 12 if H % 12 == 0 else H
    NQ = H // HCH
    R = HCH // 4

    def body(xa_ref, w1_ref, b1_ref, w2_ref, b2_ref, fw1_ref,
             fb1_ref, fw2_ref, fb2_ref, o_ref, h1_ref, h2_ref, prev_ref):
        q = pl.program_id(1)
        w1v = w1_ref[...]
        b1v = b1_ref[...]

        def c1_dot(rhs, oh, ow):
            acc = jnp.dot(w1v, rhs.reshape(K1, tb),
                          preferred_element_type=jnp.float32)
            h1_ref[oh, ow, :, :] = jnp.maximum(acc + b1v, 0.0).astype(
                jnp.bfloat16)

        for r in range(R - 1):
            for ow in range(OW1):
                c1_dot(xa_ref[:, 4 * r:4 * r + 8, pl.ds(4 * ow, 8), :],
                       R * q + r, ow)

        if NQ > 1:
            @pl.when(q > 0)
            def _boundary():
                for ow in range(OW1):
                    rhs = jnp.concatenate(
                        [prev_ref[:, 4:8, pl.ds(4 * ow, 8), :],
                         xa_ref[:, 0:4, pl.ds(4 * ow, 8), :]], axis=1)
                    c1_dot(rhs, R * q - 1, ow)

            prev_ref[...] = xa_ref[:, HCH - 8:, :, :]

        @pl.when(q == NQ - 1)
        def _tail():
            w2v = w2_ref[...]
            b2v = b2_ref[...]
            for oh2 in range(OH2):
                for ow2 in range(OW2):
                    rhs = h1_ref[pl.ds(2 * oh2, 4), pl.ds(2 * ow2, 4), :, :]
                    rhs = rhs.reshape(16 * C1, tb)
                    acc = jnp.dot(w2v, rhs,
                                  preferred_element_type=jnp.float32)
                    h2_ref[oh2, ow2, :, :] = jnp.maximum(
                        acc + b2v, 0.0).astype(jnp.bfloat16)

            flat = h2_ref[...].reshape(OH2 * OW2 * C2, tb)
            h = jnp.dot(fw1_ref[...], flat,
                        preferred_element_type=jnp.float32)
            h = jnp.maximum(h + fb1_ref[...], 0.0).astype(jnp.bfloat16)
            o = jnp.dot(fw2_ref[...], h, preferred_element_type=jnp.float32)
            o_ref[0] = o + fb2_ref[...]

    def whole(a):
        return pl.BlockSpec(a.shape, lambda i, q: (0,) * a.ndim)

    return pl.pallas_call(
        body,
        grid=(B // tb, NQ),
        in_specs=[
            pl.BlockSpec((C, HCH, W, tb), lambda i, q: (0, q, 0, i)),
            whole(w1), whole(b1), whole(w2), whole(b2),
            whole(fw1), whole(fb1), whole(fw2), whole(fb2),
        ],
        out_specs=pl.BlockSpec((1, NP, tb), lambda i, q: (i, 0, 0)),
        out_shape=jax.ShapeDtypeStruct((B // tb, NP, tb), jnp.float32),
        scratch_shapes=[
            pltpu.VMEM((OH1, OW1, C1, tb), jnp.bfloat16),
            pltpu.VMEM((OH2, OW2, C2, tb), jnp.bfloat16),
            pltpu.VMEM((C, 8, W, tb), jnp.float32),
        ],
        compiler_params=pltpu.CompilerParams(
            dimension_semantics=("parallel", "arbitrary"),
            vmem_limit_bytes=60 * 1024 * 1024,
        ),
    )(x, w1, b1, w2, b2, fw1, fb1, fw2, fb2)


def kernel(x, c1_w, c1_b, c2_w, c2_b, fc1_w, fc1_b, fc2_w, fc2_b):
    B, C, H, W = x.shape
    C1 = c1_w.shape[0]
    C2 = c2_w.shape[0]
    OH1 = (H - 8) // 4 + 1
    OH2 = (OH1 - 4) // 2 + 1
    OW2 = OH2
    tb = 256 if B % 256 == 0 else B

    idx2 = np.array([c * 16 + kh * 4 + kw
                     for kh in range(4) for kw in range(4)
                     for c in range(C1)])
    w2 = c2_w[:, idx2].astype(jnp.bfloat16)

    idxf = np.array([c2 * (OH2 * OW2) + oh2 * OW2 + ow2
                     for oh2 in range(OH2) for ow2 in range(OW2)
                     for c2 in range(C2)])
    fw1 = fc1_w[idxf, :].T.astype(jnp.bfloat16)
    fb1 = fc1_b.reshape(-1, 1).astype(jnp.float32)
    fw2 = fc2_w.T.astype(jnp.bfloat16)
    fb2 = fc2_b.reshape(-1, 1).astype(jnp.float32)

    xt = jnp.transpose(x, (1, 2, 3, 0)).astype(jnp.float32)
    out = _fused_cnn(xt, c1_w.astype(jnp.float32),
                     c1_b.astype(jnp.float32),
                     w2, c2_b.astype(jnp.float32), fw1, fb1, fw2, fb2, tb=tb)
    return jnp.swapaxes(out, 1, 2).reshape(B, -1)[:, :18]
